# Initial kernel scaffold; baseline (speedup 1.0000x reference)
import jax
import jax.numpy as jnp
from jax.experimental import pallas as pl


def kernel(cue, table):
    raise NotImplementedError("write your pallas kernel here")


# speedup vs baseline: 3.2807x; 3.2807x over previous
import jax
import jax.numpy as jnp
from jax.experimental import pallas as pl

_ROWS_PER_BLOCK = 2048


def _tc_body(cue_ref, table_ref, out_ref):
    idx = cue_ref[0, 0, :]
    w = table_ref[:]
    idx2 = idx[:, None]
    out = jnp.where(idx2 == 0, w[0][None, :], w[1][None, :])
    out = jnp.where(idx2 == 2, w[2][None, :], out)
    out = jnp.where(idx2 == 3, w[3][None, :], out)
    out_ref[:] = out


def kernel(cue, table):
    B, T = cue.shape
    n = B * T
    R = _ROWS_PER_BLOCK
    assert n % R == 0
    nblk = n // R
    cue_flat = cue.reshape(nblk, 1, R).astype(jnp.int32)
    out = pl.pallas_call(
        _tc_body,
        grid=(nblk,),
        in_specs=[
            pl.BlockSpec((1, 1, R), lambda i: (i, 0, 0)),
            pl.BlockSpec((4, 128), lambda i: (0, 0)),
        ],
        out_specs=pl.BlockSpec((R, 128), lambda i: (i, 0)),
        out_shape=jax.ShapeDtypeStruct((n, 128), jnp.float32),
    )(cue_flat, table)
    return out.reshape(B, T, 128)


# speedup vs baseline: 7.9702x; 2.4295x over previous
import jax
import jax.numpy as jnp
from jax.experimental import pallas as pl

_B_BLK = 128


def _tc_body(cue_ref, table_ref, out_ref):
    idx = cue_ref[...][:, :, None]
    w = table_ref[:]
    out = jnp.where(idx == 0, w[0][None, None, :], w[1][None, None, :])
    out = jnp.where(idx == 2, w[2][None, None, :], out)
    out = jnp.where(idx == 3, w[3][None, None, :], out)
    out_ref[...] = out


def kernel(cue, table):
    B, T = cue.shape
    assert B % _B_BLK == 0
    nblk = B // _B_BLK
    out = pl.pallas_call(
        _tc_body,
        grid=(nblk,),
        in_specs=[
            pl.BlockSpec((_B_BLK, T), lambda i: (i, 0)),
            pl.BlockSpec((4, 128), lambda i: (0, 0)),
        ],
        out_specs=pl.BlockSpec((_B_BLK, T, 128), lambda i: (i, 0, 0)),
        out_shape=jax.ShapeDtypeStruct((B, T, 128), jnp.float32),
    )(cue.astype(jnp.int32), table)
    return out
